# Initial kernel scaffold; baseline (speedup 1.0000x reference)
import jax
import jax.numpy as jnp
from jax.experimental import pallas as pl


def kernel(x, edge_index, batch, emb, W1, b1, W2, b2, W3, b3, Wfc, bfc):
    raise NotImplementedError("write your pallas kernel here")


# speedup vs baseline: 6.8881x; 6.8881x over previous
import functools

import jax
import jax.numpy as jnp
from jax import lax
from jax.experimental import pallas as pl
from jax.experimental.pallas import tpu as pltpu
from jax.experimental.pallas import tpu_sc as plsc

N = 10000
D = 128
G = 64
T = 11
NP = N + 8

NC = 2
NS = 16
NW = NC * NS
K = 128
NCHUNK = 80
EPW = NCHUNK * K
RPT = 624

_mesh = plsc.VectorSubcoreMesh(core_axis_name="c", subcore_axis_name="s")


@functools.partial(
    pl.kernel,
    out_type=(jax.ShapeDtypeStruct((N,), jnp.float32),
              jax.ShapeDtypeStruct((N,), jnp.float32)),
    mesh=_mesh,
    scratch_types=[
        pltpu.VMEM((NCHUNK, 2, K), jnp.int32),
        pltpu.VMEM((K,), jnp.float32),
        pltpu.VMEM((1024,), jnp.float32),
        pltpu.VMEM_SHARED((NP,), jnp.float32),
    ],
)
def _deg_sc(idx_hbm, out0_hbm, out1_hbm, idx_v, ones_v, zb_v, acc_s):
    cid = lax.axis_index("c")
    sid = lax.axis_index("s")
    wid = sid * NC + cid

    for j in range(K // 16):
        ones_v[pl.ds(j * 16, 16)] = jnp.ones((16,), jnp.float32)

    def zfill(i, c):
        zb_v[pl.ds(i * 16, 16)] = jnp.zeros((16,), jnp.float32)
        return c
    lax.fori_loop(0, 64, zfill, 0)

    @pl.when(sid < 10)
    def _():
        pltpu.sync_copy(zb_v.at[pl.ds(0, 1000)],
                        acc_s.at[pl.ds(sid * 1000, 1000)])

    pltpu.sync_copy(idx_hbm.at[wid], idx_v)
    plsc.subcore_barrier()

    def body(j, c):
        pltpu.sync_copy(ones_v, acc_s.at[idx_v.at[j, 1]], add=True)
        return c
    lax.fori_loop(0, NCHUNK, body, 0)

    plsc.subcore_barrier()

    @pl.when(sid < 10)
    def _():
        pltpu.sync_copy(acc_s.at[pl.ds(sid * 1000, 1000)],
                        zb_v.at[pl.ds(0, 1000)])

        @pl.when(cid == 0)
        def _():
            pltpu.sync_copy(zb_v.at[pl.ds(0, 1000)],
                            out0_hbm.at[pl.ds(sid * 1000, 1000)])

        @pl.when(cid == 1)
        def _():
            pltpu.sync_copy(zb_v.at[pl.ds(0, 1000)],
                            out1_hbm.at[pl.ds(sid * 1000, 1000)])


@functools.partial(
    pl.kernel,
    out_type=jax.ShapeDtypeStruct((NC, N, D), jnp.float32),
    mesh=_mesh,
    scratch_types=[
        pltpu.VMEM((2, 2, K), jnp.int32),
        pltpu.VMEM((2, K, D), jnp.float32),
        pltpu.VMEM((8, D), jnp.float32),
        pltpu.VMEM_SHARED((NP, D), jnp.float32),
        pltpu.SemaphoreType.DMA,
        pltpu.SemaphoreType.DMA,
        pltpu.SemaphoreType.DMA,
        pltpu.SemaphoreType.DMA,
    ],
)
def _agg_sc(idx_hbm, g_hbm, out_hbm,
            idx_v, rows_v, zer_v, acc_s, si0, si1, sg0, sg1):
    cid = lax.axis_index("c")
    sid = lax.axis_index("s")
    wid = sid * NC + cid
    base = sid * RPT

    def zf(i, c):
        for j in range(D // 16):
            zer_v[i, pl.ds(j * 16, 16)] = jnp.zeros((16,), jnp.float32)
        return c
    lax.fori_loop(0, 8, zf, 0)

    def zc(t, c):
        pltpu.sync_copy(zer_v, acc_s.at[pl.ds(base + t * 8, 8)])
        return c
    lax.fori_loop(0, RPT // 8, zc, 0)

    @pl.when(sid == NS - 1)
    def _():
        pltpu.sync_copy(zer_v, acc_s.at[pl.ds(NS * RPT, 8)])
        pltpu.sync_copy(zer_v, acc_s.at[pl.ds(NS * RPT + 8, 8)])

    plsc.subcore_barrier()

    pltpu.async_copy(idx_hbm.at[wid, 0], idx_v.at[0], si0)
    pltpu.async_copy(idx_hbm.at[wid, 1], idx_v.at[1], si1)
    pltpu.make_async_copy(idx_hbm.at[wid, 0], idx_v.at[0], si0).wait()
    pltpu.async_copy(g_hbm.at[idx_v.at[0, 0]], rows_v.at[0], sg0)

    def body(t, c):
        c0 = 2 * t
        pltpu.make_async_copy(idx_hbm.at[wid, c0 + 1], idx_v.at[1],
                              si1).wait()
        pltpu.async_copy(g_hbm.at[idx_v.at[1, 0]], rows_v.at[1], sg1)

        pltpu.make_async_copy(g_hbm.at[idx_v.at[0, 0]], rows_v.at[0],
                              sg0).wait()
        pltpu.sync_copy(rows_v.at[0], acc_s.at[idx_v.at[0, 1]], add=True)

        @pl.when(c0 + 2 < NCHUNK)
        def _():
            pltpu.async_copy(idx_hbm.at[wid, c0 + 2], idx_v.at[0], si0)

        pltpu.make_async_copy(g_hbm.at[idx_v.at[1, 0]], rows_v.at[1],
                              sg1).wait()
        pltpu.sync_copy(rows_v.at[1], acc_s.at[idx_v.at[1, 1]], add=True)

        @pl.when(c0 + 2 < NCHUNK)
        def _():
            pltpu.async_copy(idx_hbm.at[wid, c0 + 3], idx_v.at[1], si1)
            pltpu.make_async_copy(idx_hbm.at[wid, c0 + 2], idx_v.at[0],
                                  si0).wait()
            pltpu.async_copy(g_hbm.at[idx_v.at[0, 0]], rows_v.at[0], sg0)
        return c
    lax.fori_loop(0, NCHUNK // 2, body, 0)

    plsc.subcore_barrier()

    def wo(t, c):
        r0 = base + t * 48
        pltpu.sync_copy(acc_s.at[pl.ds(r0, 48)],
                        rows_v.at[0, pl.ds(0, 48)])
        pltpu.sync_copy(rows_v.at[0, pl.ds(0, 48)],
                        out_hbm.at[cid, pl.ds(r0, 48)])
        return c
    lax.fori_loop(0, RPT // 48, wo, 0)

    @pl.when(sid == NS - 1)
    def _():
        pltpu.sync_copy(acc_s.at[pl.ds(NS * RPT, 16)],
                        rows_v.at[0, pl.ds(0, 16)])
        pltpu.sync_copy(rows_v.at[0, pl.ds(0, 16)],
                        out_hbm.at[cid, pl.ds(NS * RPT, 16)])


def _prep_body(x_ref, emb_ref, w1_ref, d0_ref, d1_ref, g1_ref, dinv_ref):
    deg = d0_ref[...] + d1_ref[...] + 1.0
    dinv = lax.rsqrt(deg)
    dinv_ref[...] = dinv
    t1 = jnp.dot(emb_ref[...], w1_ref[...], preferred_element_type=jnp.float32)
    ids = lax.broadcasted_iota(jnp.int32, (N, 16), 1)
    oh = (x_ref[...] == ids).astype(jnp.float32)
    g1_ref[0:N, :] = dinv * jnp.dot(oh, t1, preferred_element_type=jnp.float32)
    g1_ref[N:NP, :] = jnp.zeros((NP - N, D), jnp.float32)


_prep_tc = pl.pallas_call(
    _prep_body,
    out_shape=(jax.ShapeDtypeStruct((NP, D), jnp.float32),
               jax.ShapeDtypeStruct((N, 1), jnp.float32)),
)


def _layer_body(a_ref, g_ref, dinv_ref, b_ref, w_ref, gn_ref):
    dinv = dinv_ref[...]
    h = jnp.maximum(
        dinv * (a_ref[0] + a_ref[1] + g_ref[0:N, :]) + b_ref[...], 0.0)
    gn_ref[0:N, :] = dinv * jnp.dot(h, w_ref[...],
                                    preferred_element_type=jnp.float32)
    gn_ref[N:NP, :] = jnp.zeros((NP - N, D), jnp.float32)


_layer_tc = pl.pallas_call(
    _layer_body,
    out_shape=jax.ShapeDtypeStruct((NP, D), jnp.float32),
)


def _final_body(a_ref, g_ref, dinv_ref, b_ref, batch_ref,
                wfc_ref, bfc_ref, out_ref):
    dinv = dinv_ref[...]
    h = jnp.maximum(
        dinv * (a_ref[0] + a_ref[1] + g_ref[0:N, :]) + b_ref[...], 0.0)
    gids = lax.broadcasted_iota(jnp.int32, (N, G), 1)
    pt = (batch_ref[...] == gids).astype(jnp.float32)
    ones = jnp.ones((N, 1), jnp.float32)
    cnt = lax.dot_general(pt, ones, (((0,), (0,)), ((), ())),
                          preferred_element_type=jnp.float32)
    ps = lax.dot_general(pt, h, (((0,), (0,)), ((), ())),
                         preferred_element_type=jnp.float32)
    pooled = ps / jnp.maximum(cnt, 1.0)
    out_ref[...] = jnp.dot(pooled, wfc_ref[...],
                           preferred_element_type=jnp.float32) + bfc_ref[...]


_final_tc = pl.pallas_call(
    _final_body,
    out_shape=jax.ShapeDtypeStruct((G, D), jnp.float32),
)


def kernel(x, edge_index, batch, emb, W1, b1, W2, b2, W3, b3, Wfc, bfc):
    npad = NW * EPW - edge_index.shape[1]
    pad = jnp.full((2, npad), N, jnp.int32)
    eidx = jnp.concatenate([edge_index.astype(jnp.int32), pad], axis=1)
    e2 = jnp.stack([eidx[0].reshape(NW, NCHUNK, K),
                    eidx[1].reshape(NW, NCHUNK, K)], axis=2)
    x2 = x.astype(jnp.int32).reshape(N, 1)
    bat2 = batch.astype(jnp.int32).reshape(N, 1)
    emb16 = jnp.zeros((16, D), jnp.float32).at[:T].set(emb)

    deg0, deg1 = _deg_sc(e2)
    g1, dinv = _prep_tc(x2, emb16, W1,
                        deg0.reshape(N, 1), deg1.reshape(N, 1))

    a = _agg_sc(e2, g1)
    g2 = _layer_tc(a, g1, dinv, b1.reshape(1, D), W2)
    a = _agg_sc(e2, g2)
    g3 = _layer_tc(a, g2, dinv, b2.reshape(1, D), W3)
    a = _agg_sc(e2, g3)
    return _final_tc(a, g3, dinv, b3.reshape(1, D), bat2,
                     Wfc, bfc.reshape(1, D))
